# baseline (speedup 1.0000x reference)
import functools

import jax
import jax.numpy as jnp
from jax import lax
from jax.experimental import pallas as pl
from jax.experimental.pallas import tpu as pltpu
from jax.experimental.pallas import tpu_sc as plsc

N = 1_000_000
D = 16
L = 16
NW = 32
C = 2048
NCHUNK = N // C
G = C // L
STEPS = (NCHUNK + NW - 1) // NW
TAIL = N - NCHUNK * C
TAIL_G = TAIL // L
TAIL_LEN = (D + 2) * TAIL
OUT_ROW = 640

_mesh = plsc.VectorSubcoreMesh(core_axis_name="c", subcore_axis_name="s")


@functools.partial(
    pl.kernel,
    mesh=_mesh,
    out_type=jax.ShapeDtypeStruct((NW * OUT_ROW,), jnp.float32),
    scratch_types=[
        pltpu.VMEM((2, D, C), jnp.float32),
        pltpu.VMEM((2, C), jnp.float32),
        pltpu.VMEM((2, C), jnp.float32),
        pltpu.VMEM((TAIL_LEN,), jnp.float32),
        pltpu.VMEM((OUT_ROW,), jnp.float32),
        pltpu.SemaphoreType.DMA,
        pltpu.SemaphoreType.DMA,
    ],
    compiler_params=pltpu.CompilerParams(use_tc_tiling_on_sc=False),
)
def _partial_moments(pos_hbm, w_hbm, l_hbm, tail_hbm, out_hbm,
                     pos_v, w_v, l_v, tail_v, out_v, sem0, sem1):
    cid = lax.axis_index("c")
    sid = lax.axis_index("s")
    wid = sid * 2 + cid
    sems = (sem0, sem1)
    zeros = jnp.zeros((L,), jnp.float32)
    ones = jnp.ones((L,), jnp.float32)

    def start(step, slot):
        c = jnp.minimum(wid + step * NW, NCHUNK - 1)
        base = c * C
        sem = sems[slot]
        handles = [
            pltpu.async_copy(w_hbm.at[pl.ds(base, C)], w_v.at[slot], sem),
            pltpu.async_copy(l_hbm.at[pl.ds(base, C)], l_v.at[slot], sem),
        ]
        for d in range(D):
            handles.append(pltpu.async_copy(
                pos_hbm.at[pl.ds(d * N + base, C)], pos_v.at[slot, d], sem))
        return handles

    def chunk_accumulate(slot, scale, accs):
        def body(g, carry):
            s0, sw2, s1, s2 = carry
            b = g * L
            o = w_v[slot, pl.ds(b, L)] * l_v[slot, pl.ds(b, L)] * scale
            s0 = s0 + o
            sw2 = sw2 + o * o
            s1n = []
            s2n = []
            for d in range(D):
                x = pos_v[slot, d, pl.ds(b, L)]
                xo = x * o
                s1n.append(s1[d] + xo)
                s2n.append(s2[d] + xo * x)
            return (s0, sw2, tuple(s1n), tuple(s2n))
        return lax.fori_loop(0, G, body, accs, unroll=False)

    accs = (zeros, zeros,
            tuple(zeros for _ in range(D)), tuple(zeros for _ in range(D)))

    pending = start(0, 0)
    for k in range(STEPS):
        slot = k % 2
        for h in pending:
            h.wait()
        nxt = start(k + 1, 1 - slot) if k + 1 < STEPS else ()
        scale = jnp.where(wid + k * NW < NCHUNK, ones, zeros)
        accs = chunk_accumulate(slot, scale, accs)
        pending = nxt

    pltpu.sync_copy(tail_hbm, tail_v)
    tail_scale = jnp.where(wid == NW - 1, ones, zeros)

    def tail_body(g, carry):
        s0, sw2, s1, s2 = carry
        b = g * L
        o = (tail_v[pl.ds(D * TAIL + b, L)]
             * tail_v[pl.ds((D + 1) * TAIL + b, L)] * tail_scale)
        s0 = s0 + o
        sw2 = sw2 + o * o
        s1n = []
        s2n = []
        for d in range(D):
            x = tail_v[pl.ds(d * TAIL + b, L)]
            xo = x * o
            s1n.append(s1[d] + xo)
            s2n.append(s2[d] + xo * x)
        return (s0, sw2, tuple(s1n), tuple(s2n))

    accs = lax.fori_loop(0, TAIL_G, tail_body, accs, unroll=False)

    s0, sw2, s1, s2 = accs
    out_v[pl.ds(0, L)] = s0
    out_v[pl.ds(L, L)] = sw2
    for d in range(D):
        out_v[pl.ds((2 + d) * L, L)] = s1[d]
        out_v[pl.ds((2 + D + d) * L, L)] = s2[d]
    for r in range(2 + 2 * D, OUT_ROW // L):
        out_v[pl.ds(r * L, L)] = zeros
    pltpu.sync_copy(out_v, out_hbm.at[pl.ds(wid * OUT_ROW, OUT_ROW)])


def kernel(positions, weights, likelihood):
    base = NCHUNK * C
    tail = jnp.concatenate([
        positions[:, base:].reshape(-1),
        weights[base:], likelihood[base:],
    ])
    part = _partial_moments(positions.reshape(-1), weights, likelihood, tail)
    part = part.reshape(NW, OUT_ROW // L, L)[:, :2 + 2 * D, :]
    sums = jnp.sum(part, axis=(0, 2))
    s0 = sums[0]
    sw2 = sums[1]
    s1 = sums[2:2 + D]
    s2 = sums[2 + D:2 + 2 * D]

    eps = jnp.finfo(jnp.float32).eps
    denom = jnp.where(jnp.isclose(s0, 0.0), s0 + eps, s0)
    mean = s1 / denom
    sw = s0 / denom
    ex2 = s2 / denom
    wss = sw2 / (denom * denom)
    nf = 1.0 / (1.0 - wss + eps)
    var = nf * (ex2 - mean * mean * (2.0 - sw))
    std = jnp.sqrt(jnp.maximum(var, 0.0))
    return jnp.stack([mean, std], axis=0)


# speedup vs baseline: 18.5739x; 18.5739x over previous
import functools

import jax
import jax.numpy as jnp
from jax import lax
from jax.experimental import pallas as pl
from jax.experimental.pallas import tpu as pltpu
from jax.experimental.pallas import tpu_sc as plsc

N = 1_000_000
D = 16
L = 16
NW = 32
C = 2048
NCHUNK = N // C
G = C // L
STEPS = (NCHUNK + NW - 1) // NW
TAIL = N - NCHUNK * C
TAIL_G = TAIL // L
TAIL_LEN = (D + 2) * TAIL
OUT_ROW = 640

_mesh = plsc.VectorSubcoreMesh(core_axis_name="c", subcore_axis_name="s")


@functools.partial(
    pl.kernel,
    mesh=_mesh,
    out_type=jax.ShapeDtypeStruct((NW * OUT_ROW,), jnp.float32),
    scratch_types=[
        pltpu.VMEM((2, D, C), jnp.float32),
        pltpu.VMEM((2, C), jnp.float32),
        pltpu.VMEM((2, C), jnp.float32),
        pltpu.VMEM((TAIL_LEN,), jnp.float32),
        pltpu.VMEM((OUT_ROW,), jnp.float32),
        pltpu.SemaphoreType.DMA,
        pltpu.SemaphoreType.DMA,
    ],
)
def _partial_moments(pos_hbm, w_hbm, l_hbm, tail_hbm, out_hbm,
                     pos_v, w_v, l_v, tail_v, out_v, sem0, sem1):
    cid = lax.axis_index("c")
    sid = lax.axis_index("s")
    wid = sid * 2 + cid
    sems = (sem0, sem1)
    zeros = jnp.zeros((L,), jnp.float32)
    ones = jnp.ones((L,), jnp.float32)

    def start(step, slot):
        c = jnp.minimum(wid + step * NW, NCHUNK - 1)
        base = c * C
        sem = sems[slot]
        handles = [
            pltpu.async_copy(w_hbm.at[pl.ds(base, C)], w_v.at[slot], sem),
            pltpu.async_copy(l_hbm.at[pl.ds(base, C)], l_v.at[slot], sem),
            pltpu.async_copy(pos_hbm.at[:, pl.ds(base, C)], pos_v.at[slot], sem),
        ]
        return handles

    def chunk_accumulate(slot, scale, accs):
        def body(g, carry):
            s0, sw2, s1, s2 = carry
            b = g * L
            o = w_v[slot, pl.ds(b, L)] * l_v[slot, pl.ds(b, L)] * scale
            s0 = s0 + o
            sw2 = sw2 + o * o
            s1n = []
            s2n = []
            for d in range(D):
                x = pos_v[slot, d, pl.ds(b, L)]
                xo = x * o
                s1n.append(s1[d] + xo)
                s2n.append(s2[d] + xo * x)
            return (s0, sw2, tuple(s1n), tuple(s2n))
        return lax.fori_loop(0, G, body, accs, unroll=False)

    accs = (zeros, zeros,
            tuple(zeros for _ in range(D)), tuple(zeros for _ in range(D)))

    pending = start(0, 0)
    for k in range(STEPS):
        slot = k % 2
        for h in pending:
            h.wait()
        nxt = start(k + 1, 1 - slot) if k + 1 < STEPS else ()
        scale = jnp.where(wid + k * NW < NCHUNK, ones, zeros)
        accs = chunk_accumulate(slot, scale, accs)
        pending = nxt

    pltpu.sync_copy(tail_hbm, tail_v)
    tail_scale = jnp.where(wid == NW - 1, ones, zeros)

    def tail_body(g, carry):
        s0, sw2, s1, s2 = carry
        b = g * L
        o = (tail_v[pl.ds(D * TAIL + b, L)]
             * tail_v[pl.ds((D + 1) * TAIL + b, L)] * tail_scale)
        s0 = s0 + o
        sw2 = sw2 + o * o
        s1n = []
        s2n = []
        for d in range(D):
            x = tail_v[pl.ds(d * TAIL + b, L)]
            xo = x * o
            s1n.append(s1[d] + xo)
            s2n.append(s2[d] + xo * x)
        return (s0, sw2, tuple(s1n), tuple(s2n))

    accs = lax.fori_loop(0, TAIL_G, tail_body, accs, unroll=False)

    s0, sw2, s1, s2 = accs
    out_v[pl.ds(0, L)] = s0
    out_v[pl.ds(L, L)] = sw2
    for d in range(D):
        out_v[pl.ds((2 + d) * L, L)] = s1[d]
        out_v[pl.ds((2 + D + d) * L, L)] = s2[d]
    for r in range(2 + 2 * D, OUT_ROW // L):
        out_v[pl.ds(r * L, L)] = zeros
    pltpu.sync_copy(out_v, out_hbm.at[pl.ds(wid * OUT_ROW, OUT_ROW)])


def kernel(positions, weights, likelihood):
    base = NCHUNK * C
    tail = jnp.concatenate([
        positions[:, base:].reshape(-1),
        weights[base:], likelihood[base:],
    ])
    part = _partial_moments(positions, weights, likelihood, tail)
    part = part.reshape(NW, OUT_ROW // L, L)[:, :2 + 2 * D, :]
    sums = jnp.sum(part, axis=(0, 2))
    s0 = sums[0]
    sw2 = sums[1]
    s1 = sums[2:2 + D]
    s2 = sums[2 + D:2 + 2 * D]

    eps = jnp.finfo(jnp.float32).eps
    denom = jnp.where(jnp.isclose(s0, 0.0), s0 + eps, s0)
    mean = s1 / denom
    sw = s0 / denom
    ex2 = s2 / denom
    wss = sw2 / (denom * denom)
    nf = 1.0 / (1.0 - wss + eps)
    var = nf * (ex2 - mean * mean * (2.0 - sw))
    std = jnp.sqrt(jnp.maximum(var, 0.0))
    return jnp.stack([mean, std], axis=0)
